# Initial kernel scaffold; baseline (speedup 1.0000x reference)
import jax
import jax.numpy as jnp
from jax.experimental import pallas as pl


def kernel(x, edge_index, W0, b0, W1, b1, W2, b2):
    raise NotImplementedError("write your pallas kernel here")


# speedup vs baseline: 32.1028x; 32.1028x over previous
import functools

import jax
import jax.numpy as jnp
from jax import lax
from jax.experimental import pallas as pl
from jax.experimental.pallas import tpu as pltpu
from jax.experimental.pallas import tpu_sc as plsc

N_NODES = 10000
D_FEAT = 128
HIDDEN = 64
NEG_SLOPE = 0.01
N_EDGES = 320000

NC = 2
NS = 16
L = 16
NW = NC * NS

CHUNK = 128
CH_PER_W = 80
E_PAD = NW * CH_PER_W * CHUNK
N_PAD = 10112
ROWS_PER_TILE = N_PAD // NS
HALF = N_PAD // 2
NV = CH_PER_W * CHUNK // L
HV = HALF // L

_MESH = plsc.VectorSubcoreMesh(core_axis_name="c", subcore_axis_name="s")
_SC_PARAMS = pltpu.CompilerParams(use_tc_tiling_on_sc=False,
                                  needs_layout_passes=False)


def _worker_ids():
    cid = lax.axis_index("c")
    sid = lax.axis_index("s")
    return cid, sid, sid * NC + cid


def _deg_body(dst_hbm, out_hbm, dst_v, hist_v, red_v):
    cid, sid, wid = _worker_ids()
    pltpu.sync_copy(dst_hbm.at[wid], dst_v)
    lanes = lax.iota(jnp.int32, L)
    ones16 = jnp.full((L,), 1.0, jnp.float32)
    zeros16 = jnp.zeros((L,), jnp.float32)

    for h in range(2):
        lo = h * HALF

        def zb(i, c):
            for r in range(L):
                hist_v[r, pl.ds(i * L, L)] = zeros16
            return c

        lax.fori_loop(0, HV, zb, 0)

        def cb(i, c):
            idx = dst_v[i // 8, pl.ds((i % 8) * L, L)]
            m = (idx >= lo) & (idx < lo + HALF)
            col = jnp.where(m, idx - lo, 0)
            plsc.addupdate_scatter(hist_v, [lanes, col], ones16, mask=m)
            return c

        lax.fori_loop(0, NV, cb, 0)

        def rb(i, c):
            s = hist_v[0, pl.ds(i * L, L)]
            for r in range(1, L):
                s = s + hist_v[r, pl.ds(i * L, L)]
            red_v[pl.ds(lo + i * L, L)] = s
            return c

        lax.fori_loop(0, HV, rb, 0)

    pltpu.sync_copy(red_v, out_hbm.at[wid])


_deg = functools.partial(
    pl.kernel,
    out_type=jax.ShapeDtypeStruct((NW, N_PAD), jnp.float32),
    mesh=_MESH,
    compiler_params=_SC_PARAMS,
    scratch_types=[
        pltpu.VMEM((CH_PER_W, CHUNK), jnp.int32),
        pltpu.VMEM((L, HALF), jnp.float32),
        pltpu.VMEM((N_PAD,), jnp.float32),
    ],
)(_deg_body)


def _agg_body(u_hbm, src_hbm, dst_hbm, zeros_hbm, out_hbm,
              src_v, dst_v, rows_v, agg_sh, sem0, sem1):
    cid, sid, wid = _worker_ids()
    base = sid * ROWS_PER_TILE
    pltpu.sync_copy(src_hbm.at[wid], src_v)
    pltpu.sync_copy(dst_hbm.at[wid], dst_v)
    pltpu.sync_copy(zeros_hbm.at[pl.ds(base, ROWS_PER_TILE)],
                    agg_sh.at[pl.ds(base, ROWS_PER_TILE)])
    plsc.subcore_barrier()

    sems = (sem0, sem1)
    pltpu.async_copy(u_hbm.at[src_v.at[0]], rows_v.at[0], sem0)

    def body(g, carry):
        for b in range(2):
            j = g * 2 + b
            nb = (b + 1) % 2

            @pl.when(j + 1 < CH_PER_W)
            def _():
                pltpu.async_copy(u_hbm.at[src_v.at[j + 1]], rows_v.at[nb],
                                 sems[nb])

            pltpu.make_async_copy(u_hbm.at[src_v.at[j]], rows_v.at[b],
                                  sems[b]).wait()
            pltpu.sync_copy(rows_v.at[b], agg_sh.at[dst_v.at[j]], add=True)
        return carry

    lax.fori_loop(0, CH_PER_W // 2, body, 0)
    plsc.subcore_barrier()
    pltpu.sync_copy(agg_sh.at[pl.ds(base, ROWS_PER_TILE)],
                    out_hbm.at[cid, pl.ds(base, ROWS_PER_TILE)])


_agg = functools.partial(
    pl.kernel,
    out_type=jax.ShapeDtypeStruct((NC, N_PAD, HIDDEN), jnp.float32),
    mesh=_MESH,
    compiler_params=_SC_PARAMS,
    scratch_types=[
        pltpu.VMEM((CH_PER_W, CHUNK), jnp.int32),
        pltpu.VMEM((CH_PER_W, CHUNK), jnp.int32),
        pltpu.VMEM((2, CHUNK, HIDDEN), jnp.float32),
        pltpu.VMEM_SHARED((N_PAD, HIDDEN), jnp.float32),
        pltpu.SemaphoreType.DMA,
        pltpu.SemaphoreType.DMA,
    ],
)(_agg_body)


def _tc_first_body(deg_ref, ones_ref, x_ref, w_ref, u_ref, dinv_ref):
    deg = lax.dot_general(deg_ref[...], ones_ref[...],
                          (((0,), (0,)), ((), ())),
                          preferred_element_type=jnp.float32)
    dinv = lax.rsqrt(deg + 1.0)
    h = jnp.dot(x_ref[...], w_ref[...], preferred_element_type=jnp.float32)
    u_ref[...] = h * dinv
    dinv_ref[...] = dinv


_tc_first = pl.pallas_call(
    _tc_first_body,
    out_shape=[
        jax.ShapeDtypeStruct((N_PAD, HIDDEN), jnp.float32),
        jax.ShapeDtypeStruct((N_PAD, 1), jnp.float32),
    ],
)


def _tc_mid_body(agg_ref, u_ref, dinv_ref, b_ref, w_ref, unext_ref):
    a = agg_ref[...]
    dinv = dinv_ref[...]
    z = (a[0] + a[1] + u_ref[...]) * dinv + b_ref[...]
    h = jnp.where(z >= 0, z, NEG_SLOPE * z)
    unext_ref[...] = jnp.dot(h, w_ref[...],
                             preferred_element_type=jnp.float32) * dinv


_tc_mid = pl.pallas_call(
    _tc_mid_body,
    out_shape=jax.ShapeDtypeStruct((N_PAD, HIDDEN), jnp.float32),
)


def _tc_last_body(agg_ref, u_ref, dinv_ref, b_ref, out_ref):
    a = agg_ref[...]
    z = (a[0] + a[1] + u_ref[...]) * dinv_ref[...] + b_ref[...]
    out_ref[...] = jnp.where(z >= 0, z, NEG_SLOPE * z)


_tc_last = pl.pallas_call(
    _tc_last_body,
    out_shape=jax.ShapeDtypeStruct((N_PAD, HIDDEN), jnp.float32),
)


def kernel(x, edge_index, W0, b0, W1, b1, W2, b2):
    src = edge_index[0]
    dst = edge_index[1]
    pad_ids = (jnp.arange(E_PAD - N_EDGES, dtype=jnp.int32) % L) + N_NODES
    src_p = jnp.concatenate([src, pad_ids]).reshape(NW, CH_PER_W, CHUNK)
    dst_p = jnp.concatenate([dst, pad_ids]).reshape(NW, CH_PER_W, CHUNK)
    x_p = jnp.pad(x, ((0, N_PAD - N_NODES), (0, 0)))

    ones_nw = jnp.ones((NW, 1), jnp.float32)
    zeros_h = jnp.zeros((N_PAD, HIDDEN), jnp.float32)
    b0r = b0.reshape(1, HIDDEN)
    b1r = b1.reshape(1, HIDDEN)
    b2r = b2.reshape(1, HIDDEN)

    deg_parts = _deg(dst_p)
    u0, dinv = _tc_first(deg_parts, ones_nw, x_p, W0)
    agg0 = _agg(u0, src_p, dst_p, zeros_h)
    u1 = _tc_mid(agg0, u0, dinv, b0r, W1)
    agg1 = _agg(u1, src_p, dst_p, zeros_h)
    u2 = _tc_mid(agg1, u1, dinv, b1r, W2)
    agg2 = _agg(u2, src_p, dst_p, zeros_h)
    out = _tc_last(agg2, u2, dinv, b2r)
    return out[:N_NODES]


# speedup vs baseline: 33.9190x; 1.0566x over previous
import functools

import jax
import jax.numpy as jnp
from jax import lax
from jax.experimental import pallas as pl
from jax.experimental.pallas import tpu as pltpu
from jax.experimental.pallas import tpu_sc as plsc

N_NODES = 10000
D_FEAT = 128
HIDDEN = 64
NEG_SLOPE = 0.01
N_EDGES = 320000

NC = 2
NS = 16
L = 16
NW = NC * NS

CHUNK = 128
CH_PER_W = 80
E_PAD = NW * CH_PER_W * CHUNK
N_PAD = 10112
ROWS_PER_TILE = N_PAD // NS
HALF = N_PAD // 2
NV = CH_PER_W * CHUNK // L
HV = HALF // L

_MESH = plsc.VectorSubcoreMesh(core_axis_name="c", subcore_axis_name="s")
_SC_PARAMS = pltpu.CompilerParams(use_tc_tiling_on_sc=False,
                                  needs_layout_passes=False)


def _worker_ids():
    cid = lax.axis_index("c")
    sid = lax.axis_index("s")
    return cid, sid, sid * NC + cid


def _deg_body(dst_hbm, out_hbm, dst_v, hist_v, red_v):
    cid, sid, wid = _worker_ids()
    pltpu.sync_copy(dst_hbm.at[wid], dst_v)
    lanes = lax.iota(jnp.int32, L)
    ones16 = jnp.full((L,), 1.0, jnp.float32)
    zeros16 = jnp.zeros((L,), jnp.float32)

    for h in range(2):
        lo = h * HALF

        def zb(i, c):
            for r in range(L):
                hist_v[r, pl.ds(i * L, L)] = zeros16
            return c

        lax.fori_loop(0, HV, zb, 0)

        def cb(i, c):
            idx = dst_v[i // 8, pl.ds((i % 8) * L, L)]
            m = (idx >= lo) & (idx < lo + HALF)
            col = jnp.where(m, idx - lo, 0)
            plsc.addupdate_scatter(hist_v, [lanes, col], ones16, mask=m)
            return c

        lax.fori_loop(0, NV, cb, 0)

        def rb(i, c):
            s = hist_v[0, pl.ds(i * L, L)]
            for r in range(1, L):
                s = s + hist_v[r, pl.ds(i * L, L)]
            red_v[pl.ds(lo + i * L, L)] = s
            return c

        lax.fori_loop(0, HV, rb, 0)

    pltpu.sync_copy(red_v, out_hbm.at[wid])


_deg = functools.partial(
    pl.kernel,
    out_type=jax.ShapeDtypeStruct((NW, N_PAD), jnp.float32),
    mesh=_MESH,
    compiler_params=_SC_PARAMS,
    scratch_types=[
        pltpu.VMEM((CH_PER_W, CHUNK), jnp.int32),
        pltpu.VMEM((L, HALF), jnp.float32),
        pltpu.VMEM((N_PAD,), jnp.float32),
    ],
)(_deg_body)


NBUF = 4


def _agg_body(u_hbm, src_hbm, dst_hbm, zeros_hbm, out_hbm,
              src_v, dst_v, rows_v, agg_sh,
              zsem, g0, g1, g2, g3, s0, s1, s2, s3):
    cid, sid, wid = _worker_ids()
    base = sid * ROWS_PER_TILE
    pltpu.sync_copy(src_hbm.at[wid], src_v)
    pltpu.sync_copy(dst_hbm.at[wid], dst_v)
    pltpu.async_copy(zeros_hbm.at[pl.ds(base, ROWS_PER_TILE)],
                     agg_sh.at[pl.ds(base, ROWS_PER_TILE)], zsem)
    gs = (g0, g1, g2, g3)
    ss = (s0, s1, s2, s3)
    for b in range(NBUF - 1):
        pltpu.async_copy(u_hbm.at[src_v.at[b]], rows_v.at[b], gs[b])
    pltpu.make_async_copy(zeros_hbm.at[pl.ds(base, ROWS_PER_TILE)],
                          agg_sh.at[pl.ds(base, ROWS_PER_TILE)], zsem).wait()
    plsc.subcore_barrier()

    def body(g, carry):
        for b in range(NBUF):
            j = g * NBUF + b
            bb = (b + NBUF - 1) % NBUF

            @pl.when((j >= 1) & (j + NBUF - 1 < CH_PER_W))
            def _():
                pltpu.make_async_copy(rows_v.at[bb],
                                      agg_sh.at[dst_v.at[j - 1]],
                                      ss[bb]).wait()
                pltpu.async_copy(u_hbm.at[src_v.at[j + NBUF - 1]],
                                 rows_v.at[bb], gs[bb])

            @pl.when(j == 0)
            def _():
                pltpu.async_copy(u_hbm.at[src_v.at[NBUF - 1]],
                                 rows_v.at[NBUF - 1], gs[NBUF - 1])

            pltpu.make_async_copy(u_hbm.at[src_v.at[j]], rows_v.at[b],
                                  gs[b]).wait()
            pltpu.async_copy(rows_v.at[b], agg_sh.at[dst_v.at[j]], ss[b],
                             add=True)
        return carry

    lax.fori_loop(0, CH_PER_W // NBUF, body, 0)
    for b in range(NBUF):
        pltpu.make_async_copy(rows_v.at[b],
                              agg_sh.at[dst_v.at[CH_PER_W - NBUF + b]],
                              ss[b]).wait()
    plsc.subcore_barrier()
    pltpu.sync_copy(agg_sh.at[pl.ds(base, ROWS_PER_TILE)],
                    out_hbm.at[cid, pl.ds(base, ROWS_PER_TILE)])


_agg = functools.partial(
    pl.kernel,
    out_type=jax.ShapeDtypeStruct((NC, N_PAD, HIDDEN), jnp.float32),
    mesh=_MESH,
    compiler_params=_SC_PARAMS,
    scratch_types=[
        pltpu.VMEM((CH_PER_W, CHUNK), jnp.int32),
        pltpu.VMEM((CH_PER_W, CHUNK), jnp.int32),
        pltpu.VMEM((NBUF, CHUNK, HIDDEN), jnp.float32),
        pltpu.VMEM_SHARED((N_PAD, HIDDEN), jnp.float32),
        pltpu.SemaphoreType.DMA,
        pltpu.SemaphoreType.DMA,
        pltpu.SemaphoreType.DMA,
        pltpu.SemaphoreType.DMA,
        pltpu.SemaphoreType.DMA,
        pltpu.SemaphoreType.DMA,
        pltpu.SemaphoreType.DMA,
        pltpu.SemaphoreType.DMA,
        pltpu.SemaphoreType.DMA,
    ],
)(_agg_body)


def _tc_first_body(deg_ref, ones_ref, x_ref, w_ref, u_ref, dinv_ref):
    deg = lax.dot_general(deg_ref[...], ones_ref[...],
                          (((0,), (0,)), ((), ())),
                          preferred_element_type=jnp.float32)
    dinv = lax.rsqrt(deg + 1.0)
    h = jnp.dot(x_ref[...], w_ref[...], preferred_element_type=jnp.float32)
    u_ref[...] = h * dinv
    dinv_ref[...] = dinv


_tc_first = pl.pallas_call(
    _tc_first_body,
    out_shape=[
        jax.ShapeDtypeStruct((N_PAD, HIDDEN), jnp.float32),
        jax.ShapeDtypeStruct((N_PAD, 1), jnp.float32),
    ],
)


def _tc_mid_body(agg_ref, u_ref, dinv_ref, b_ref, w_ref, unext_ref):
    a = agg_ref[...]
    dinv = dinv_ref[...]
    z = (a[0] + a[1] + u_ref[...]) * dinv + b_ref[...]
    h = jnp.where(z >= 0, z, NEG_SLOPE * z)
    unext_ref[...] = jnp.dot(h, w_ref[...],
                             preferred_element_type=jnp.float32) * dinv


_tc_mid = pl.pallas_call(
    _tc_mid_body,
    out_shape=jax.ShapeDtypeStruct((N_PAD, HIDDEN), jnp.float32),
)


def _tc_last_body(agg_ref, u_ref, dinv_ref, b_ref, out_ref):
    a = agg_ref[...]
    z = (a[0] + a[1] + u_ref[...]) * dinv_ref[...] + b_ref[...]
    out_ref[...] = jnp.where(z >= 0, z, NEG_SLOPE * z)


_tc_last = pl.pallas_call(
    _tc_last_body,
    out_shape=jax.ShapeDtypeStruct((N_PAD, HIDDEN), jnp.float32),
)


def kernel(x, edge_index, W0, b0, W1, b1, W2, b2):
    src = edge_index[0]
    dst = edge_index[1]
    pad_ids = (jnp.arange(E_PAD - N_EDGES, dtype=jnp.int32) % L) + N_NODES
    src_p = jnp.concatenate([src, pad_ids]).reshape(NW, CH_PER_W, CHUNK)
    dst_p = jnp.concatenate([dst, pad_ids]).reshape(NW, CH_PER_W, CHUNK)
    x_p = jnp.pad(x, ((0, N_PAD - N_NODES), (0, 0)))

    ones_nw = jnp.ones((NW, 1), jnp.float32)
    zeros_h = jnp.zeros((N_PAD, HIDDEN), jnp.float32)
    b0r = b0.reshape(1, HIDDEN)
    b1r = b1.reshape(1, HIDDEN)
    b2r = b2.reshape(1, HIDDEN)

    deg_parts = _deg(dst_p)
    u0, dinv = _tc_first(deg_parts, ones_nw, x_p, W0)
    agg0 = _agg(u0, src_p, dst_p, zeros_h)
    u1 = _tc_mid(agg0, u0, dinv, b0r, W1)
    agg1 = _agg(u1, src_p, dst_p, zeros_h)
    u2 = _tc_mid(agg1, u1, dinv, b1r, W2)
    agg2 = _agg(u2, src_p, dst_p, zeros_h)
    out = _tc_last(agg2, u2, dinv, b2r)
    return out[:N_NODES]


# speedup vs baseline: 39.0082x; 1.1500x over previous
import functools

import jax
import jax.numpy as jnp
from jax import lax
from jax.experimental import pallas as pl
from jax.experimental.pallas import tpu as pltpu
from jax.experimental.pallas import tpu_sc as plsc

N_NODES = 10000
D_FEAT = 128
HIDDEN = 64
NEG_SLOPE = 0.01
N_EDGES = 320000

NC = 2
NS = 16
L = 16
NW = NC * NS

CHUNK = 128
CH_PER_W = 80
E_PAD = NW * CH_PER_W * CHUNK
N_PAD = 10112
ROWS_PER_TILE = N_PAD // NS
HALF = N_PAD // 2
NV = CH_PER_W * CHUNK // L
HV = HALF // L

_MESH = plsc.VectorSubcoreMesh(core_axis_name="c", subcore_axis_name="s")
_SC_PARAMS = pltpu.CompilerParams(use_tc_tiling_on_sc=False,
                                  needs_layout_passes=False)


def _worker_ids():
    cid = lax.axis_index("c")
    sid = lax.axis_index("s")
    return cid, sid, sid * NC + cid


def _deg_body(dst_hbm, out_hbm, dst_v, hist_v, red_v):
    cid, sid, wid = _worker_ids()
    pltpu.sync_copy(dst_hbm.at[wid], dst_v)
    lanes = lax.iota(jnp.int32, L)
    ones16 = jnp.full((L,), 1.0, jnp.float32)
    zeros16 = jnp.zeros((L,), jnp.float32)

    for h in range(2):
        lo = h * HALF

        def zb(i, c):
            for r in range(L):
                hist_v[r, pl.ds(i * L, L)] = zeros16
            return c

        lax.fori_loop(0, HV, zb, 0)

        def cb(i, c):
            idx = dst_v[i // 8, pl.ds((i % 8) * L, L)]
            m = (idx >= lo) & (idx < lo + HALF)
            col = jnp.where(m, idx - lo, 0)
            plsc.addupdate_scatter(hist_v, [lanes, col], ones16, mask=m)
            return c

        lax.fori_loop(0, NV, cb, 0)

        def rb(i, c):
            s = hist_v[0, pl.ds(i * L, L)]
            for r in range(1, L):
                s = s + hist_v[r, pl.ds(i * L, L)]
            red_v[pl.ds(lo + i * L, L)] = s
            return c

        lax.fori_loop(0, HV, rb, 0)

    pltpu.sync_copy(red_v, out_hbm.at[wid])


_deg = functools.partial(
    pl.kernel,
    out_type=jax.ShapeDtypeStruct((NW, N_PAD), jnp.float32),
    mesh=_MESH,
    compiler_params=_SC_PARAMS,
    scratch_types=[
        pltpu.VMEM((CH_PER_W, CHUNK), jnp.int32),
        pltpu.VMEM((L, HALF), jnp.float32),
        pltpu.VMEM((N_PAD,), jnp.float32),
    ],
)(_deg_body)


NBUF = 4


def _agg_body(u_hbm, src_hbm, dst_hbm, zeros_hbm, out_hbm,
              src_v, dst_v, rows_v, agg_sh,
              zsem, g0, g1, g2, g3, s0, s1, s2, s3):
    cid, sid, wid = _worker_ids()
    base = sid * ROWS_PER_TILE
    pltpu.sync_copy(src_hbm.at[wid], src_v)
    pltpu.sync_copy(dst_hbm.at[wid], dst_v)
    pltpu.async_copy(zeros_hbm.at[pl.ds(base, ROWS_PER_TILE)],
                     agg_sh.at[pl.ds(base, ROWS_PER_TILE)], zsem)
    gs = (g0, g1, g2, g3)
    ss = (s0, s1, s2, s3)
    for b in range(NBUF - 1):
        pltpu.async_copy(u_hbm.at[src_v.at[b]], rows_v.at[b], gs[b])
    pltpu.make_async_copy(zeros_hbm.at[pl.ds(base, ROWS_PER_TILE)],
                          agg_sh.at[pl.ds(base, ROWS_PER_TILE)], zsem).wait()
    plsc.subcore_barrier()

    def body(g, carry):
        for b in range(NBUF):
            j = g * NBUF + b
            bb = (b + NBUF - 1) % NBUF

            @pl.when((j >= 1) & (j + NBUF - 1 < CH_PER_W))
            def _():
                pltpu.make_async_copy(rows_v.at[bb],
                                      agg_sh.at[dst_v.at[j - 1]],
                                      ss[bb]).wait()
                pltpu.async_copy(u_hbm.at[src_v.at[j + NBUF - 1]],
                                 rows_v.at[bb], gs[bb])

            @pl.when(j == 0)
            def _():
                pltpu.async_copy(u_hbm.at[src_v.at[NBUF - 1]],
                                 rows_v.at[NBUF - 1], gs[NBUF - 1])

            pltpu.make_async_copy(u_hbm.at[src_v.at[j]], rows_v.at[b],
                                  gs[b]).wait()
            pltpu.async_copy(rows_v.at[b], agg_sh.at[dst_v.at[j]], ss[b],
                             add=True)
        return carry

    lax.fori_loop(0, CH_PER_W // NBUF, body, 0)
    for b in range(NBUF):
        pltpu.make_async_copy(rows_v.at[b],
                              agg_sh.at[dst_v.at[CH_PER_W - NBUF + b]],
                              ss[b]).wait()
    plsc.subcore_barrier()
    pltpu.sync_copy(agg_sh.at[pl.ds(base, ROWS_PER_TILE)],
                    out_hbm.at[cid, pl.ds(base, ROWS_PER_TILE)])


_agg = functools.partial(
    pl.kernel,
    out_type=jax.ShapeDtypeStruct((NC, N_PAD, HIDDEN), jnp.float32),
    mesh=_MESH,
    compiler_params=_SC_PARAMS,
    scratch_types=[
        pltpu.VMEM((CH_PER_W, CHUNK), jnp.int32),
        pltpu.VMEM((CH_PER_W, CHUNK), jnp.int32),
        pltpu.VMEM((NBUF, CHUNK, HIDDEN), jnp.float32),
        pltpu.VMEM_SHARED((N_PAD, HIDDEN), jnp.float32),
        pltpu.SemaphoreType.DMA,
        pltpu.SemaphoreType.DMA,
        pltpu.SemaphoreType.DMA,
        pltpu.SemaphoreType.DMA,
        pltpu.SemaphoreType.DMA,
        pltpu.SemaphoreType.DMA,
        pltpu.SemaphoreType.DMA,
        pltpu.SemaphoreType.DMA,
        pltpu.SemaphoreType.DMA,
    ],
)(_agg_body)


def _tc_first_body(deg_ref, ones_ref, x_ref, w_ref, u_ref, dinv_ref):
    deg = lax.dot_general(deg_ref[...], ones_ref[...],
                          (((0,), (0,)), ((), ())),
                          preferred_element_type=jnp.float32)
    dinv = lax.rsqrt(deg + 1.0)
    h = jnp.dot(x_ref[...], w_ref[...], preferred_element_type=jnp.float32)
    u_ref[...] = h * dinv
    dinv_ref[...] = dinv


_tc_first = pl.pallas_call(
    _tc_first_body,
    out_shape=[
        jax.ShapeDtypeStruct((N_PAD, HIDDEN), jnp.float32),
        jax.ShapeDtypeStruct((N_PAD, 1), jnp.float32),
    ],
)


def _tc_mid_body(agg_ref, u_ref, dinv_ref, b_ref, w_ref, unext_ref):
    a = agg_ref[...]
    dinv = dinv_ref[...]
    z = (a[0] + a[1] + u_ref[...]) * dinv + b_ref[...]
    h = jnp.where(z >= 0, z, NEG_SLOPE * z)
    unext_ref[...] = jnp.dot(h, w_ref[...],
                             preferred_element_type=jnp.float32) * dinv


_tc_mid = pl.pallas_call(
    _tc_mid_body,
    out_shape=jax.ShapeDtypeStruct((N_PAD, HIDDEN), jnp.float32),
)


def _tc_last_body(agg_ref, u_ref, dinv_ref, b_ref, out_ref):
    a = agg_ref[...]
    z = (a[0] + a[1] + u_ref[...]) * dinv_ref[...] + b_ref[...]
    out_ref[...] = jnp.where(z >= 0, z, NEG_SLOPE * z)


_tc_last = pl.pallas_call(
    _tc_last_body,
    out_shape=jax.ShapeDtypeStruct((N_PAD, HIDDEN), jnp.float32),
)


def kernel(x, edge_index, W0, b0, W1, b1, W2, b2):
    src = edge_index[0]
    dst = edge_index[1]
    pad_per_w = (E_PAD - N_EDGES) // NW
    real_per_w = N_EDGES // NW
    pad_ids = (jnp.arange(E_PAD - N_EDGES, dtype=jnp.int32)
               % (N_PAD - N_NODES)) + N_NODES
    pad_block = pad_ids.reshape(NW, pad_per_w)
    src_p = jnp.concatenate([src.reshape(NW, real_per_w), pad_block],
                            axis=1).reshape(NW, CH_PER_W, CHUNK)
    dst_p = jnp.concatenate([dst.reshape(NW, real_per_w), pad_block],
                            axis=1).reshape(NW, CH_PER_W, CHUNK)
    x_p = jnp.pad(x, ((0, N_PAD - N_NODES), (0, 0)))

    ones_nw = jnp.ones((NW, 1), jnp.float32)
    zeros_h = jnp.zeros((N_PAD, HIDDEN), jnp.float32)
    b0r = b0.reshape(1, HIDDEN)
    b1r = b1.reshape(1, HIDDEN)
    b2r = b2.reshape(1, HIDDEN)

    deg_parts = _deg(dst_p)
    u0, dinv = _tc_first(deg_parts, ones_nw, x_p, W0)
    agg0 = _agg(u0, src_p, dst_p, zeros_h)
    u1 = _tc_mid(agg0, u0, dinv, b0r, W1)
    agg1 = _agg(u1, src_p, dst_p, zeros_h)
    u2 = _tc_mid(agg1, u1, dinv, b1r, W2)
    agg2 = _agg(u2, src_p, dst_p, zeros_h)
    out = _tc_last(agg2, u2, dinv, b2r)
    return out[:N_NODES]


# speedup vs baseline: 41.1396x; 1.0546x over previous
import functools

import jax
import jax.numpy as jnp
from jax import lax
from jax.experimental import pallas as pl
from jax.experimental.pallas import tpu as pltpu
from jax.experimental.pallas import tpu_sc as plsc

N_NODES = 10000
D_FEAT = 128
HIDDEN = 64
NEG_SLOPE = 0.01
N_EDGES = 320000

NC = 2
NS = 16
L = 16
NW = NC * NS

CHUNK = 128
CH_PER_W = 80
E_PAD = NW * CH_PER_W * CHUNK
N_PAD = 10112
ROWS_PER_TILE = N_PAD // NS
HALF = N_PAD // 2
NV = CH_PER_W * CHUNK // L
HV = HALF // L

_MESH = plsc.VectorSubcoreMesh(core_axis_name="c", subcore_axis_name="s")
_SC_PARAMS = pltpu.CompilerParams(use_tc_tiling_on_sc=False,
                                  needs_layout_passes=False)


def _worker_ids():
    cid = lax.axis_index("c")
    sid = lax.axis_index("s")
    return cid, sid, sid * NC + cid


def _deg_body(dst_hbm, out_hbm, dst_v, red_v):
    cid, sid, wid = _worker_ids()
    pltpu.sync_copy(dst_hbm.at[wid], dst_v)
    ones16 = jnp.full((L,), 1.0, jnp.float32)
    zeros16 = jnp.zeros((L,), jnp.float32)

    def zb(i, c):
        red_v[pl.ds(i * L, L)] = zeros16
        return c

    lax.fori_loop(0, N_PAD // L, zb, 0)

    def cb(i, c):
        idx = dst_v[i // 8, pl.ds((i % 8) * L, L)]
        plsc.addupdate_scatter(red_v, [idx], ones16)
        return c

    lax.fori_loop(0, NV, cb, 0)
    pltpu.sync_copy(red_v, out_hbm.at[wid])


_deg = functools.partial(
    pl.kernel,
    out_type=jax.ShapeDtypeStruct((NW, N_PAD), jnp.float32),
    mesh=_MESH,
    compiler_params=_SC_PARAMS,
    scratch_types=[
        pltpu.VMEM((CH_PER_W, CHUNK), jnp.int32),
        pltpu.VMEM((N_PAD,), jnp.float32),
    ],
)(_deg_body)


NBUF = 4


def _agg_body(u_hbm, src_hbm, dst_hbm, zeros_hbm, out_hbm,
              src_v, dst_v, rows_v, agg_sh,
              zsem, g0, g1, g2, g3, s0, s1, s2, s3):
    cid, sid, wid = _worker_ids()
    base = sid * ROWS_PER_TILE
    pltpu.sync_copy(src_hbm.at[wid], src_v)
    pltpu.sync_copy(dst_hbm.at[wid], dst_v)
    pltpu.async_copy(zeros_hbm.at[pl.ds(base, ROWS_PER_TILE)],
                     agg_sh.at[pl.ds(base, ROWS_PER_TILE)], zsem)
    gs = (g0, g1, g2, g3)
    ss = (s0, s1, s2, s3)
    for b in range(NBUF - 1):
        pltpu.async_copy(u_hbm.at[src_v.at[b]], rows_v.at[b], gs[b])
    pltpu.make_async_copy(zeros_hbm.at[pl.ds(base, ROWS_PER_TILE)],
                          agg_sh.at[pl.ds(base, ROWS_PER_TILE)], zsem).wait()
    plsc.subcore_barrier()

    def body(g, carry):
        for b in range(NBUF):
            j = g * NBUF + b
            bb = (b + NBUF - 1) % NBUF

            @pl.when((j >= 1) & (j + NBUF - 1 < CH_PER_W))
            def _():
                pltpu.make_async_copy(rows_v.at[bb],
                                      agg_sh.at[dst_v.at[j - 1]],
                                      ss[bb]).wait()
                pltpu.async_copy(u_hbm.at[src_v.at[j + NBUF - 1]],
                                 rows_v.at[bb], gs[bb])

            @pl.when(j == 0)
            def _():
                pltpu.async_copy(u_hbm.at[src_v.at[NBUF - 1]],
                                 rows_v.at[NBUF - 1], gs[NBUF - 1])

            pltpu.make_async_copy(u_hbm.at[src_v.at[j]], rows_v.at[b],
                                  gs[b]).wait()
            pltpu.async_copy(rows_v.at[b], agg_sh.at[dst_v.at[j]], ss[b],
                             add=True)
        return carry

    lax.fori_loop(0, CH_PER_W // NBUF, body, 0)
    for b in range(NBUF):
        pltpu.make_async_copy(rows_v.at[b],
                              agg_sh.at[dst_v.at[CH_PER_W - NBUF + b]],
                              ss[b]).wait()
    plsc.subcore_barrier()
    pltpu.sync_copy(agg_sh.at[pl.ds(base, ROWS_PER_TILE)],
                    out_hbm.at[cid, pl.ds(base, ROWS_PER_TILE)])


_agg = functools.partial(
    pl.kernel,
    out_type=jax.ShapeDtypeStruct((NC, N_PAD, HIDDEN), jnp.float32),
    mesh=_MESH,
    compiler_params=_SC_PARAMS,
    scratch_types=[
        pltpu.VMEM((CH_PER_W, CHUNK), jnp.int32),
        pltpu.VMEM((CH_PER_W, CHUNK), jnp.int32),
        pltpu.VMEM((NBUF, CHUNK, HIDDEN), jnp.float32),
        pltpu.VMEM_SHARED((N_PAD, HIDDEN), jnp.float32),
        pltpu.SemaphoreType.DMA,
        pltpu.SemaphoreType.DMA,
        pltpu.SemaphoreType.DMA,
        pltpu.SemaphoreType.DMA,
        pltpu.SemaphoreType.DMA,
        pltpu.SemaphoreType.DMA,
        pltpu.SemaphoreType.DMA,
        pltpu.SemaphoreType.DMA,
        pltpu.SemaphoreType.DMA,
    ],
)(_agg_body)


def _tc_first_body(deg_ref, ones_ref, x_ref, w_ref, u_ref, dinv_ref):
    deg = lax.dot_general(deg_ref[...], ones_ref[...],
                          (((0,), (0,)), ((), ())),
                          preferred_element_type=jnp.float32)
    dinv = lax.rsqrt(deg + 1.0)
    h = jnp.dot(x_ref[...], w_ref[...], preferred_element_type=jnp.float32)
    u_ref[...] = h * dinv
    dinv_ref[...] = dinv


_tc_first = pl.pallas_call(
    _tc_first_body,
    out_shape=[
        jax.ShapeDtypeStruct((N_PAD, HIDDEN), jnp.float32),
        jax.ShapeDtypeStruct((N_PAD, 1), jnp.float32),
    ],
)


def _tc_mid_body(agg_ref, u_ref, dinv_ref, b_ref, w_ref, unext_ref):
    a = agg_ref[...]
    dinv = dinv_ref[...]
    z = (a[0] + a[1] + u_ref[...]) * dinv + b_ref[...]
    h = jnp.where(z >= 0, z, NEG_SLOPE * z)
    unext_ref[...] = jnp.dot(h, w_ref[...],
                             preferred_element_type=jnp.float32) * dinv


_tc_mid = pl.pallas_call(
    _tc_mid_body,
    out_shape=jax.ShapeDtypeStruct((N_PAD, HIDDEN), jnp.float32),
)


def _tc_last_body(agg_ref, u_ref, dinv_ref, b_ref, out_ref):
    a = agg_ref[...]
    z = (a[0] + a[1] + u_ref[...]) * dinv_ref[...] + b_ref[...]
    out_ref[...] = jnp.where(z >= 0, z, NEG_SLOPE * z)


_tc_last = pl.pallas_call(
    _tc_last_body,
    out_shape=jax.ShapeDtypeStruct((N_PAD, HIDDEN), jnp.float32),
)


def kernel(x, edge_index, W0, b0, W1, b1, W2, b2):
    src = edge_index[0]
    dst = edge_index[1]
    pad_per_w = (E_PAD - N_EDGES) // NW
    real_per_w = N_EDGES // NW
    pad_ids = (jnp.arange(E_PAD - N_EDGES, dtype=jnp.int32)
               % (N_PAD - N_NODES)) + N_NODES
    pad_block = pad_ids.reshape(NW, pad_per_w)
    src_p = jnp.concatenate([src.reshape(NW, real_per_w), pad_block],
                            axis=1).reshape(NW, CH_PER_W, CHUNK)
    dst_p = jnp.concatenate([dst.reshape(NW, real_per_w), pad_block],
                            axis=1).reshape(NW, CH_PER_W, CHUNK)
    x_p = jnp.pad(x, ((0, N_PAD - N_NODES), (0, 0)))

    ones_nw = jnp.ones((NW, 1), jnp.float32)
    zeros_h = jnp.zeros((N_PAD, HIDDEN), jnp.float32)
    b0r = b0.reshape(1, HIDDEN)
    b1r = b1.reshape(1, HIDDEN)
    b2r = b2.reshape(1, HIDDEN)

    deg_parts = _deg(dst_p)
    u0, dinv = _tc_first(deg_parts, ones_nw, x_p, W0)
    agg0 = _agg(u0, src_p, dst_p, zeros_h)
    u1 = _tc_mid(agg0, u0, dinv, b0r, W1)
    agg1 = _agg(u1, src_p, dst_p, zeros_h)
    u2 = _tc_mid(agg1, u1, dinv, b1r, W2)
    agg2 = _agg(u2, src_p, dst_p, zeros_h)
    out = _tc_last(agg2, u2, dinv, b2r)
    return out[:N_NODES]
